# Initial kernel scaffold; baseline (speedup 1.0000x reference)
import jax
import jax.numpy as jnp
from jax.experimental import pallas as pl


def kernel(x, from_indices, to_indices, reactances, limits):
    raise NotImplementedError("write your pallas kernel here")


# speedup vs baseline: 2.8420x; 2.8420x over previous
import functools

import jax
import jax.numpy as jnp
from jax import lax
from jax.experimental import pallas as pl
from jax.experimental.pallas import tpu as pltpu
from jax.experimental.pallas import tpu_sc as plsc

N_BUSES = 10000
N_LINES = 160000
N_BATCH = 64
LANES = 16
CHUNK = 8000
N_CHUNKS = N_LINES // CHUNK
VECS = CHUNK // LANES
ROWS_PER_TILE = 2


def _sc_kernel(angles_hbm, fi_hbm, ti_hbm, rc_hbm, lm_hbm,
               ang2_out, flows_out,
               ang, ang2, fi, ti, rc, lm, fbuf):
    c = lax.axis_index("c")
    s = lax.axis_index("s")
    wid = s * 2 + c

    for rr in range(ROWS_PER_TILE):
        row = wid * ROWS_PER_TILE + rr
        arow = row * N_BUSES
        frow = row * N_LINES
        pltpu.sync_copy(angles_hbm.at[pl.ds(arow, N_BUSES)], ang)
        pltpu.sync_copy(angles_hbm.at[pl.ds(arow, N_BUSES)], ang2)

        def chunk1(ci, _):
            base = ci * CHUNK
            pltpu.sync_copy(fi_hbm.at[pl.ds(base, CHUNK)], fi)
            pltpu.sync_copy(ti_hbm.at[pl.ds(base, CHUNK)], ti)
            pltpu.sync_copy(rc_hbm.at[pl.ds(base, CHUNK)], rc)
            pltpu.sync_copy(lm_hbm.at[pl.ds(base, CHUNK)], lm)

            def vec1(vi, _):
                o = vi * LANES
                fidx = fi[pl.ds(o, LANES)]
                tidx = ti[pl.ds(o, LANES)]
                fa = plsc.load_gather(ang, [fidx])
                ta = plsc.load_gather(ang, [tidx])
                d = fa - ta
                rl = rc[pl.ds(o, LANES)] * lm[pl.ds(o, LANES)]
                adj = jnp.where(jnp.abs(d) > rl,
                                (jnp.sign(d) * rl - d) * 0.5,
                                jnp.zeros_like(d))
                plsc.addupdate_scatter(ang2, [fidx], adj)
                plsc.addupdate_scatter(ang2, [tidx], adj)
                return _

            lax.fori_loop(0, VECS, vec1, None)
            return _

        lax.fori_loop(0, N_CHUNKS, chunk1, None)

        def chunk2(ci, _):
            base = ci * CHUNK
            pltpu.sync_copy(fi_hbm.at[pl.ds(base, CHUNK)], fi)
            pltpu.sync_copy(ti_hbm.at[pl.ds(base, CHUNK)], ti)
            pltpu.sync_copy(rc_hbm.at[pl.ds(base, CHUNK)], rc)

            def vec2(vi, _):
                o = vi * LANES
                fidx = fi[pl.ds(o, LANES)]
                tidx = ti[pl.ds(o, LANES)]
                fa = plsc.load_gather(ang2, [fidx])
                ta = plsc.load_gather(ang2, [tidx])
                fbuf[pl.ds(o, LANES)] = (fa - ta) / rc[pl.ds(o, LANES)]
                return _

            lax.fori_loop(0, VECS, vec2, None)
            pltpu.sync_copy(fbuf, flows_out.at[pl.ds(frow + base, CHUNK)])
            return _

        lax.fori_loop(0, N_CHUNKS, chunk2, None)
        pltpu.sync_copy(ang2, ang2_out.at[pl.ds(arow, N_BUSES)])


@jax.jit
def _run(angles, from_indices, to_indices, reactances, limits):
    mesh = plsc.VectorSubcoreMesh(core_axis_name="c", subcore_axis_name="s")
    f = functools.partial(
        pl.kernel,
        mesh=mesh,
        compiler_params=pltpu.CompilerParams(needs_layout_passes=False),
        out_type=[
            jax.ShapeDtypeStruct((N_BATCH * N_BUSES,), jnp.float32),
            jax.ShapeDtypeStruct((N_BATCH * N_LINES,), jnp.float32),
        ],
        scratch_types=[
            pltpu.VMEM((N_BUSES,), jnp.float32),
            pltpu.VMEM((N_BUSES,), jnp.float32),
            pltpu.VMEM((CHUNK,), jnp.int32),
            pltpu.VMEM((CHUNK,), jnp.int32),
            pltpu.VMEM((CHUNK,), jnp.float32),
            pltpu.VMEM((CHUNK,), jnp.float32),
            pltpu.VMEM((CHUNK,), jnp.float32),
        ],
    )(_sc_kernel)
    return f(angles, from_indices, to_indices, reactances, limits)


def kernel(x, from_indices, to_indices, reactances, limits):
    angles = x[:, N_BUSES:2 * N_BUSES].reshape(-1)
    angles2, flows2 = _run(
        angles,
        from_indices.astype(jnp.int32),
        to_indices.astype(jnp.int32),
        reactances,
        limits,
    )
    angles2 = angles2.reshape(N_BATCH, N_BUSES)
    flows2 = flows2.reshape(N_BATCH, N_LINES)
    out = jnp.concatenate(
        [x[:, :N_BUSES], angles2, x[:, 2 * N_BUSES:]], axis=1)
    return (out, flows2)


# speedup vs baseline: 3.5656x; 1.2546x over previous
import functools

import jax
import jax.numpy as jnp
from jax import lax
from jax.experimental import pallas as pl
from jax.experimental.pallas import tpu as pltpu
from jax.experimental.pallas import tpu_sc as plsc

N_BUSES = 10000
N_LINES = 160000
N_BATCH = 64
LANES = 16
CHUNK = 8000
N_CHUNKS = N_LINES // CHUNK
VECS = CHUNK // LANES
ROWS_PER_TILE = 2


def _sc_kernel(angles_hbm, fi_hbm, ti_hbm, rl_hbm, ir_hbm,
               ang2_out, flows_out,
               ang_a, ang_b, ang2_a, ang2_b, fi, ti, rbuf, fbuf_a, fbuf_b):
    c = lax.axis_index("c")
    s = lax.axis_index("s")
    wid = s * 2 + c
    row_a = wid * ROWS_PER_TILE
    row_b = row_a + 1

    pltpu.sync_copy(angles_hbm.at[pl.ds(row_a * N_BUSES, N_BUSES)], ang_a)
    pltpu.sync_copy(angles_hbm.at[pl.ds(row_b * N_BUSES, N_BUSES)], ang_b)
    pltpu.sync_copy(angles_hbm.at[pl.ds(row_a * N_BUSES, N_BUSES)], ang2_a)
    pltpu.sync_copy(angles_hbm.at[pl.ds(row_b * N_BUSES, N_BUSES)], ang2_b)

    def chunk1(ci, _):
        base = ci * CHUNK
        pltpu.sync_copy(fi_hbm.at[pl.ds(base, CHUNK)], fi)
        pltpu.sync_copy(ti_hbm.at[pl.ds(base, CHUNK)], ti)
        pltpu.sync_copy(rl_hbm.at[pl.ds(base, CHUNK)], rbuf)

        def vec1(vi, _):
            o = vi * LANES
            fidx = fi[pl.ds(o, LANES)]
            tidx = ti[pl.ds(o, LANES)]
            rl = rbuf[pl.ds(o, LANES)]
            for ang, ang2 in ((ang_a, ang2_a), (ang_b, ang2_b)):
                fa = plsc.load_gather(ang, [fidx])
                ta = plsc.load_gather(ang, [tidx])
                d = fa - ta
                adj = jnp.where(jnp.abs(d) > rl,
                                (jnp.sign(d) * rl - d) * 0.5,
                                jnp.zeros_like(d))
                plsc.addupdate_scatter(ang2, [fidx], adj)
                plsc.addupdate_scatter(ang2, [tidx], adj)
            return _

        lax.fori_loop(0, VECS, vec1, None)
        return _

    lax.fori_loop(0, N_CHUNKS, chunk1, None)

    def chunk2(ci, _):
        base = ci * CHUNK
        pltpu.sync_copy(fi_hbm.at[pl.ds(base, CHUNK)], fi)
        pltpu.sync_copy(ti_hbm.at[pl.ds(base, CHUNK)], ti)
        pltpu.sync_copy(ir_hbm.at[pl.ds(base, CHUNK)], rbuf)

        def vec2(vi, _):
            o = vi * LANES
            fidx = fi[pl.ds(o, LANES)]
            tidx = ti[pl.ds(o, LANES)]
            ir = rbuf[pl.ds(o, LANES)]
            for ang2, fbuf in ((ang2_a, fbuf_a), (ang2_b, fbuf_b)):
                fa = plsc.load_gather(ang2, [fidx])
                ta = plsc.load_gather(ang2, [tidx])
                fbuf[pl.ds(o, LANES)] = (fa - ta) * ir
            return _

        lax.fori_loop(0, VECS, vec2, None)
        pltpu.sync_copy(fbuf_a, flows_out.at[pl.ds(row_a * N_LINES + base, CHUNK)])
        pltpu.sync_copy(fbuf_b, flows_out.at[pl.ds(row_b * N_LINES + base, CHUNK)])
        return _

    lax.fori_loop(0, N_CHUNKS, chunk2, None)
    pltpu.sync_copy(ang2_a, ang2_out.at[pl.ds(row_a * N_BUSES, N_BUSES)])
    pltpu.sync_copy(ang2_b, ang2_out.at[pl.ds(row_b * N_BUSES, N_BUSES)])


@jax.jit
def _run(angles, from_indices, to_indices, rl, inv_r):
    mesh = plsc.VectorSubcoreMesh(core_axis_name="c", subcore_axis_name="s")
    f = functools.partial(
        pl.kernel,
        mesh=mesh,
        compiler_params=pltpu.CompilerParams(needs_layout_passes=False),
        out_type=[
            jax.ShapeDtypeStruct((N_BATCH * N_BUSES,), jnp.float32),
            jax.ShapeDtypeStruct((N_BATCH * N_LINES,), jnp.float32),
        ],
        scratch_types=[
            pltpu.VMEM((N_BUSES,), jnp.float32),
            pltpu.VMEM((N_BUSES,), jnp.float32),
            pltpu.VMEM((N_BUSES,), jnp.float32),
            pltpu.VMEM((N_BUSES,), jnp.float32),
            pltpu.VMEM((CHUNK,), jnp.int32),
            pltpu.VMEM((CHUNK,), jnp.int32),
            pltpu.VMEM((CHUNK,), jnp.float32),
            pltpu.VMEM((CHUNK,), jnp.float32),
            pltpu.VMEM((CHUNK,), jnp.float32),
        ],
    )(_sc_kernel)
    return f(angles, from_indices, to_indices, rl, inv_r)


def kernel(x, from_indices, to_indices, reactances, limits):
    angles = x[:, N_BUSES:2 * N_BUSES].reshape(-1)
    angles2, flows2 = _run(
        angles,
        from_indices.astype(jnp.int32),
        to_indices.astype(jnp.int32),
        reactances * limits,
        1.0 / reactances,
    )
    angles2 = angles2.reshape(N_BATCH, N_BUSES)
    flows2 = flows2.reshape(N_BATCH, N_LINES)
    out = jnp.concatenate(
        [x[:, :N_BUSES], angles2, x[:, 2 * N_BUSES:]], axis=1)
    return (out, flows2)


# speedup vs baseline: 6.7869x; 1.9034x over previous
import functools

import jax
import jax.numpy as jnp
from jax import lax
from jax.experimental import pallas as pl
from jax.experimental.pallas import tpu as pltpu
from jax.experimental.pallas import tpu_sc as plsc

N_BUSES = 10000
N_LINES = 160000
N_BATCH = 64
LANES = 16
CHUNK = 8000
N_CHUNKS = N_LINES // CHUNK
VECS = CHUNK // LANES
ROWS_PER_TILE = 2


def _sc_kernel(angles_hbm, fi_hbm, ti_hbm, rl_hbm, ir_hbm,
               ang2_out, flows_out,
               ang_a, ang_b, ang2_a, ang2_b, fi, ti, rbuf, fbuf_a, fbuf_b):
    c = lax.axis_index("c")
    s = lax.axis_index("s")
    wid = s * 2 + c
    row_a = wid * ROWS_PER_TILE
    row_b = row_a + 1

    pltpu.sync_copy(angles_hbm.at[pl.ds(row_a * N_BUSES, N_BUSES)], ang_a)
    pltpu.sync_copy(angles_hbm.at[pl.ds(row_b * N_BUSES, N_BUSES)], ang_b)
    pltpu.sync_copy(angles_hbm.at[pl.ds(row_a * N_BUSES, N_BUSES)], ang2_a)
    pltpu.sync_copy(angles_hbm.at[pl.ds(row_b * N_BUSES, N_BUSES)], ang2_b)

    def chunk1(ci, _):
        base = ci * CHUNK
        pltpu.sync_copy(fi_hbm.at[pl.ds(base, CHUNK)], fi)
        pltpu.sync_copy(ti_hbm.at[pl.ds(base, CHUNK)], ti)
        pltpu.sync_copy(rl_hbm.at[pl.ds(base, CHUNK)], rbuf)

        @plsc.parallel_loop(0, CHUNK, LANES, unroll=4)
        def vec1(o):
            fidx = fi[pl.ds(o, LANES)]
            tidx = ti[pl.ds(o, LANES)]
            rl = rbuf[pl.ds(o, LANES)]
            for ang, ang2 in ((ang_a, ang2_a), (ang_b, ang2_b)):
                fa = plsc.load_gather(ang, [fidx])
                ta = plsc.load_gather(ang, [tidx])
                d = fa - ta
                adj = jnp.where(jnp.abs(d) > rl,
                                (jnp.sign(d) * rl - d) * 0.5,
                                jnp.zeros_like(d))
                plsc.addupdate_scatter(ang2, [fidx], adj)
                plsc.addupdate_scatter(ang2, [tidx], adj)

        return _

    lax.fori_loop(0, N_CHUNKS, chunk1, None)

    def chunk2(ci, _):
        base = ci * CHUNK
        pltpu.sync_copy(fi_hbm.at[pl.ds(base, CHUNK)], fi)
        pltpu.sync_copy(ti_hbm.at[pl.ds(base, CHUNK)], ti)
        pltpu.sync_copy(ir_hbm.at[pl.ds(base, CHUNK)], rbuf)

        @plsc.parallel_loop(0, CHUNK, LANES, unroll=4)
        def vec2(o):
            fidx = fi[pl.ds(o, LANES)]
            tidx = ti[pl.ds(o, LANES)]
            ir = rbuf[pl.ds(o, LANES)]
            for ang2, fbuf in ((ang2_a, fbuf_a), (ang2_b, fbuf_b)):
                fa = plsc.load_gather(ang2, [fidx])
                ta = plsc.load_gather(ang2, [tidx])
                fbuf[pl.ds(o, LANES)] = (fa - ta) * ir
        pltpu.sync_copy(fbuf_a, flows_out.at[pl.ds(row_a * N_LINES + base, CHUNK)])
        pltpu.sync_copy(fbuf_b, flows_out.at[pl.ds(row_b * N_LINES + base, CHUNK)])
        return _

    lax.fori_loop(0, N_CHUNKS, chunk2, None)
    pltpu.sync_copy(ang2_a, ang2_out.at[pl.ds(row_a * N_BUSES, N_BUSES)])
    pltpu.sync_copy(ang2_b, ang2_out.at[pl.ds(row_b * N_BUSES, N_BUSES)])


@jax.jit
def _run(angles, from_indices, to_indices, rl, inv_r):
    mesh = plsc.VectorSubcoreMesh(core_axis_name="c", subcore_axis_name="s")
    f = functools.partial(
        pl.kernel,
        mesh=mesh,
        compiler_params=pltpu.CompilerParams(needs_layout_passes=False),
        out_type=[
            jax.ShapeDtypeStruct((N_BATCH * N_BUSES,), jnp.float32),
            jax.ShapeDtypeStruct((N_BATCH * N_LINES,), jnp.float32),
        ],
        scratch_types=[
            pltpu.VMEM((N_BUSES,), jnp.float32),
            pltpu.VMEM((N_BUSES,), jnp.float32),
            pltpu.VMEM((N_BUSES,), jnp.float32),
            pltpu.VMEM((N_BUSES,), jnp.float32),
            pltpu.VMEM((CHUNK,), jnp.int32),
            pltpu.VMEM((CHUNK,), jnp.int32),
            pltpu.VMEM((CHUNK,), jnp.float32),
            pltpu.VMEM((CHUNK,), jnp.float32),
            pltpu.VMEM((CHUNK,), jnp.float32),
        ],
    )(_sc_kernel)
    return f(angles, from_indices, to_indices, rl, inv_r)


def kernel(x, from_indices, to_indices, reactances, limits):
    angles = x[:, N_BUSES:2 * N_BUSES].reshape(-1)
    angles2, flows2 = _run(
        angles,
        from_indices.astype(jnp.int32),
        to_indices.astype(jnp.int32),
        reactances * limits,
        1.0 / reactances,
    )
    angles2 = angles2.reshape(N_BATCH, N_BUSES)
    flows2 = flows2.reshape(N_BATCH, N_LINES)
    out = jnp.concatenate(
        [x[:, :N_BUSES], angles2, x[:, 2 * N_BUSES:]], axis=1)
    return (out, flows2)


# speedup vs baseline: 9.8860x; 1.4566x over previous
import functools

import jax
import jax.numpy as jnp
from jax import lax
from jax.experimental import pallas as pl
from jax.experimental.pallas import tpu as pltpu
from jax.experimental.pallas import tpu_sc as plsc

N_BUSES = 10000
N_LINES = 160000
N_BATCH = 64
LANES = 16
CHUNK = 8000
N_CHUNKS = N_LINES // CHUNK
ROWS_PER_TILE = 2
UNROLL = 4


def _sc_kernel(angles_hbm, fi_hbm, ti_hbm, rl_hbm, ir_hbm,
               ang2_out, flows_out,
               ang_a, ang_b, ang2_a, ang2_b,
               fi0, ti0, r0, fi1, ti1, r1,
               fba0, fbb0, fba1, fbb1,
               sin0, sin1, sout0, sout1):
    c = lax.axis_index("c")
    s = lax.axis_index("s")
    wid = s * 2 + c
    row_a = wid * ROWS_PER_TILE
    row_b = row_a + 1

    IN = ((fi0, ti0, r0, sin0), (fi1, ti1, r1, sin1))
    OUT = ((fba0, fbb0, sout0), (fba1, fbb1, sout1))

    def start_in(b, base, r_hbm):
        fib, tib, rb, sem = IN[b]
        pltpu.async_copy(fi_hbm.at[pl.ds(base, CHUNK)], fib, sem)
        pltpu.async_copy(ti_hbm.at[pl.ds(base, CHUNK)], tib, sem)
        pltpu.async_copy(r_hbm.at[pl.ds(base, CHUNK)], rb, sem)

    def wait_in(b):
        fib, tib, rb, sem = IN[b]
        pltpu.make_async_copy(fi_hbm.at[pl.ds(0, CHUNK)], fib, sem).wait()
        pltpu.make_async_copy(ti_hbm.at[pl.ds(0, CHUNK)], tib, sem).wait()
        pltpu.make_async_copy(rl_hbm.at[pl.ds(0, CHUNK)], rb, sem).wait()

    def start_out(b, base):
        fba, fbb, sem = OUT[b]
        pltpu.async_copy(
            fba, flows_out.at[pl.ds(row_a * N_LINES + base, CHUNK)], sem)
        pltpu.async_copy(
            fbb, flows_out.at[pl.ds(row_b * N_LINES + base, CHUNK)], sem)

    def wait_out(b):
        fba, fbb, sem = OUT[b]
        pltpu.make_async_copy(fba, flows_out.at[pl.ds(0, CHUNK)], sem).wait()
        pltpu.make_async_copy(fbb, flows_out.at[pl.ds(0, CHUNK)], sem).wait()

    pltpu.sync_copy(angles_hbm.at[pl.ds(row_a * N_BUSES, N_BUSES)], ang_a)
    pltpu.sync_copy(angles_hbm.at[pl.ds(row_b * N_BUSES, N_BUSES)], ang_b)
    pltpu.sync_copy(angles_hbm.at[pl.ds(row_a * N_BUSES, N_BUSES)], ang2_a)
    pltpu.sync_copy(angles_hbm.at[pl.ds(row_b * N_BUSES, N_BUSES)], ang2_b)

    start_in(0, 0, rl_hbm)

    @pl.loop(0, N_CHUNKS, step=2)
    def phase1(ci):
        for b in range(2):
            cur = ci + b
            wait_in(b)

            @pl.when(cur + 1 < N_CHUNKS)
            def _():
                start_in(1 - b, (cur + 1) * CHUNK, rl_hbm)

            fib, tib, rb, _sem = IN[b]

            @plsc.parallel_loop(0, CHUNK, LANES, unroll=UNROLL)
            def vec1(o):
                fidx = fib[pl.ds(o, LANES)]
                tidx = tib[pl.ds(o, LANES)]
                rl = rb[pl.ds(o, LANES)]
                for ang, ang2 in ((ang_a, ang2_a), (ang_b, ang2_b)):
                    fa = plsc.load_gather(ang, [fidx])
                    ta = plsc.load_gather(ang, [tidx])
                    d = fa - ta
                    adj = jnp.where(jnp.abs(d) > rl,
                                    (jnp.sign(d) * rl - d) * 0.5,
                                    jnp.zeros_like(d))
                    plsc.addupdate_scatter(ang2, [fidx], adj)
                    plsc.addupdate_scatter(ang2, [tidx], adj)

    start_in(0, 0, ir_hbm)

    @pl.loop(0, N_CHUNKS, step=2)
    def phase2(ci):
        for b in range(2):
            cur = ci + b
            wait_in(b)

            @pl.when(cur + 1 < N_CHUNKS)
            def _():
                start_in(1 - b, (cur + 1) * CHUNK, ir_hbm)

            @pl.when(cur >= 2)
            def _():
                wait_out(b)

            fib, tib, rb, _sem = IN[b]
            fba, fbb, _osem = OUT[b]

            @plsc.parallel_loop(0, CHUNK, LANES, unroll=UNROLL)
            def vec2(o):
                fidx = fib[pl.ds(o, LANES)]
                tidx = tib[pl.ds(o, LANES)]
                ir = rb[pl.ds(o, LANES)]
                for ang2, fbuf in ((ang2_a, fba), (ang2_b, fbb)):
                    fa = plsc.load_gather(ang2, [fidx])
                    ta = plsc.load_gather(ang2, [tidx])
                    fbuf[pl.ds(o, LANES)] = (fa - ta) * ir

            start_out(b, cur * CHUNK)

    wait_out(0)
    wait_out(1)
    pltpu.sync_copy(ang2_a, ang2_out.at[pl.ds(row_a * N_BUSES, N_BUSES)])
    pltpu.sync_copy(ang2_b, ang2_out.at[pl.ds(row_b * N_BUSES, N_BUSES)])


@jax.jit
def _run(angles, from_indices, to_indices, rl, inv_r):
    mesh = plsc.VectorSubcoreMesh(core_axis_name="c", subcore_axis_name="s")
    f = functools.partial(
        pl.kernel,
        mesh=mesh,
        compiler_params=pltpu.CompilerParams(needs_layout_passes=False),
        out_type=[
            jax.ShapeDtypeStruct((N_BATCH * N_BUSES,), jnp.float32),
            jax.ShapeDtypeStruct((N_BATCH * N_LINES,), jnp.float32),
        ],
        scratch_types=[
            pltpu.VMEM((N_BUSES,), jnp.float32),
            pltpu.VMEM((N_BUSES,), jnp.float32),
            pltpu.VMEM((N_BUSES,), jnp.float32),
            pltpu.VMEM((N_BUSES,), jnp.float32),
            pltpu.VMEM((CHUNK,), jnp.int32),
            pltpu.VMEM((CHUNK,), jnp.int32),
            pltpu.VMEM((CHUNK,), jnp.float32),
            pltpu.VMEM((CHUNK,), jnp.int32),
            pltpu.VMEM((CHUNK,), jnp.int32),
            pltpu.VMEM((CHUNK,), jnp.float32),
            pltpu.VMEM((CHUNK,), jnp.float32),
            pltpu.VMEM((CHUNK,), jnp.float32),
            pltpu.VMEM((CHUNK,), jnp.float32),
            pltpu.VMEM((CHUNK,), jnp.float32),
            pltpu.SemaphoreType.DMA,
            pltpu.SemaphoreType.DMA,
            pltpu.SemaphoreType.DMA,
            pltpu.SemaphoreType.DMA,
        ],
    )(_sc_kernel)
    return f(angles, from_indices, to_indices, rl, inv_r)


def kernel(x, from_indices, to_indices, reactances, limits):
    angles = x[:, N_BUSES:2 * N_BUSES].reshape(-1)
    angles2, flows2 = _run(
        angles,
        from_indices.astype(jnp.int32),
        to_indices.astype(jnp.int32),
        reactances * limits,
        1.0 / reactances,
    )
    angles2 = angles2.reshape(N_BATCH, N_BUSES)
    flows2 = flows2.reshape(N_BATCH, N_LINES)
    out = jnp.concatenate(
        [x[:, :N_BUSES], angles2, x[:, 2 * N_BUSES:]], axis=1)
    return (out, flows2)
